# Initial kernel scaffold; baseline (speedup 1.0000x reference)
import jax
import jax.numpy as jnp
from jax.experimental import pallas as pl


def kernel(input, adj, W):
    raise NotImplementedError("write your pallas kernel here")


# speedup vs baseline: 1.0666x; 1.0666x over previous
import jax
import jax.numpy as jnp
from jax.experimental import pallas as pl

N = 10000
F_IN = 128
F_OUT = 128
BM = 400


def _gcn_block_kernel(adj_ref, x_ref, w_ref, out_ref):
    i = pl.program_id(0)
    support = jnp.dot(adj_ref[...], x_ref[...],
                      preferred_element_type=jnp.float32)
    xb = x_ref[pl.ds(i * BM, BM), :]
    out_ref[...] = (
        jnp.dot(xb, w_ref[:F_IN, :], preferred_element_type=jnp.float32)
        + jnp.dot(support, w_ref[F_IN:, :], preferred_element_type=jnp.float32)
    )


def kernel(input, adj, W):
    return pl.pallas_call(
        _gcn_block_kernel,
        grid=(N // BM,),
        in_specs=[
            pl.BlockSpec((BM, N), lambda i: (i, 0)),
            pl.BlockSpec((N, F_IN), lambda i: (0, 0)),
            pl.BlockSpec((2 * F_IN, F_OUT), lambda i: (0, 0)),
        ],
        out_specs=pl.BlockSpec((BM, F_OUT), lambda i: (i, 0)),
        out_shape=jax.ShapeDtypeStruct((N, F_OUT), jnp.float32),
    )(adj, input, W)
